# Initial kernel scaffold; baseline (speedup 1.0000x reference)
import jax
import jax.numpy as jnp
from jax.experimental import pallas as pl


def kernel(x, edge_index, edge_indice, edge_type, edge_dialog, W1, A1, b1, W2, A2, b2):
    raise NotImplementedError("write your pallas kernel here")


# speedup vs baseline: 27.6412x; 27.6412x over previous
import functools

import jax
import jax.numpy as jnp
from jax import lax
from jax.experimental import pallas as pl
from jax.experimental.pallas import tpu as pltpu
from jax.experimental.pallas import tpu_sc as plsc

N = 10000
E = 320000
D = 128
H = 4
DH = 32

NC = 2
NS = 16
NW = NC * NS
K = 128
EW = 10112
NB = EW // K
EPAD = NW * EW
NACC = N + 1

_f32 = jnp.float32
_i32 = jnp.int32


def _prep_body(h_ref, wc_ref, ss_ref, sd_ref, xf_ref, as_ref, ad_ref):
    xfb = jax.lax.dot_general(
        h_ref[...], wc_ref[...], (((1,), (0,)), ((), ())),
        precision=jax.lax.Precision.HIGHEST, preferred_element_type=_f32)
    xf_ref[...] = xfb
    as_ref[...] = jax.lax.dot_general(
        xfb, ss_ref[...], (((1,), (0,)), ((), ())),
        precision=jax.lax.Precision.HIGHEST, preferred_element_type=_f32)
    ad_ref[...] = jax.lax.dot_general(
        xfb, sd_ref[...], (((1,), (0,)), ((), ())),
        precision=jax.lax.Precision.HIGHEST, preferred_element_type=_f32)


def _prep(h, wcat, ssrc, sdst):
    bm = 400
    grid = (N // bm,)
    return pl.pallas_call(
        _prep_body,
        grid=grid,
        in_specs=[
            pl.BlockSpec((bm, D), lambda i: (i, 0)),
            pl.BlockSpec((D, D), lambda i: (0, 0)),
            pl.BlockSpec((D, 16), lambda i: (0, 0)),
            pl.BlockSpec((D, 16), lambda i: (0, 0)),
        ],
        out_specs=[
            pl.BlockSpec((bm, D), lambda i: (i, 0)),
            pl.BlockSpec((bm, 16), lambda i: (i, 0)),
            pl.BlockSpec((bm, 16), lambda i: (i, 0)),
        ],
        out_shape=[
            jax.ShapeDtypeStruct((N, D), _f32),
            jax.ShapeDtypeStruct((N, 16), _f32),
            jax.ShapeDtypeStruct((N, 16), _f32),
        ],
    )(h, wcat, ssrc, sdst)


def _edge_body(xf_hbm, as_hbm, ad_hbm, src_hbm, dst_hbm, g_hbm, zn_hbm,
               zd_hbm, pnum_hbm, pden_hbm,
               acc_n, acc_d, src_i, dst_i, a_s, a_d, w_r, xfr, g_v):
    c = lax.axis_index("c")
    s = lax.axis_index("s")
    wid = c * NS + s

    iota = lax.iota(_i32, 16)
    headmask = iota < H
    zeros16 = jnp.zeros((16,), _f32)

    nz = (NACC + K - 1) // K
    for k in range(5):
        t = s + NS * k

        @pl.when(t < nz)
        def _():
            start = jnp.minimum(t * K, NACC - K)
            pltpu.sync_copy(zn_hbm, acc_n.at[pl.ds(start, K)])
            pltpu.sync_copy(zd_hbm, acc_d.at[pl.ds(start, K)])

    pltpu.sync_copy(g_hbm, g_v)
    plsc.subcore_barrier()

    gv = g_v[...]

    def _block(b, _):
        base = wid * EW + b * K
        pltpu.sync_copy(src_hbm.at[pl.ds(base, K)], src_i)
        pltpu.sync_copy(dst_hbm.at[pl.ds(base, K)], dst_i)
        pltpu.sync_copy(xf_hbm.at[src_i], xfr)
        pltpu.sync_copy(as_hbm.at[src_i], a_s)
        pltpu.sync_copy(ad_hbm.at[dst_i], a_d)

        def _edge(e, _):
            asv = a_s[e, :]
            adv = a_d[e, :]
            sv = asv + adv
            lr = jnp.maximum(sv, 0.2 * sv)
            tv = gv + adv
            mv = jnp.maximum(tv, 0.2 * tv)
            wv = jnp.where(headmask, jnp.exp(lr - mv), zeros16)
            w_r[e, :] = wv
            for h in range(H):
                wb = wv.at[jnp.full((16,), h, _i32)].get(
                    mode="promise_in_bounds")
                for half in range(2):
                    j = (h * 2 + half) * 16
                    xfr[e, pl.ds(j, 16)] = xfr[e, pl.ds(j, 16)] * wb
            return 0
        lax.fori_loop(0, K, _edge, 0)

        pltpu.sync_copy(xfr, acc_n.at[dst_i], add=True)
        pltpu.sync_copy(w_r, acc_d.at[dst_i], add=True)
        return 0

    lax.fori_loop(0, NB, _block, 0)
    plsc.subcore_barrier()

    nd = (N + K - 1) // K
    for k in range(5):
        t = s + NS * k

        @pl.when(t < nd)
        def _():
            start = jnp.minimum(t * K, N - K)
            pltpu.sync_copy(acc_n.at[pl.ds(start, K)],
                            pnum_hbm.at[c, pl.ds(start, K)])
            pltpu.sync_copy(acc_d.at[pl.ds(start, K)],
                            pden_hbm.at[c, pl.ds(start, K)])


@functools.cache
def _edge_sc_build():
    return pl.kernel(
        _edge_body,
        out_type=[
            jax.ShapeDtypeStruct((NC, N, D), _f32),
            jax.ShapeDtypeStruct((NC, N, 16), _f32),
        ],
        mesh=plsc.VectorSubcoreMesh(core_axis_name="c", subcore_axis_name="s",
                                    num_cores=NC, num_subcores=NS),
        compiler_params=pltpu.CompilerParams(use_tc_tiling_on_sc=False),
        scratch_types=[
            pltpu.VMEM_SHARED((NACC, D), _f32),
            pltpu.VMEM_SHARED((NACC, 16), _f32),
            pltpu.VMEM((K,), _i32),
            pltpu.VMEM((K,), _i32),
            pltpu.VMEM((K, 16), _f32),
            pltpu.VMEM((K, 16), _f32),
            pltpu.VMEM((K, 16), _f32),
            pltpu.VMEM((K, D), _f32),
            pltpu.VMEM((16,), _f32),
        ],
    )


def _comb_body(pn_ref, pd_ref, rep_ref, b_ref, h_ref, out_ref):
    num = pn_ref[0] + pn_ref[1]
    den = pd_ref[0] + pd_ref[1]
    denx = jax.lax.dot_general(
        den, rep_ref[...], (((1,), (0,)), ((), ())),
        precision=jax.lax.Precision.HIGHEST, preferred_element_type=_f32)
    att = num / (denx + 1e-16) + b_ref[...]
    out_ref[...] = h_ref[...] + jnp.where(
        att > 0, att, jnp.exp(jnp.minimum(att, 0.0)) - 1.0)


def _combine(pnum, pden, rep, b2d, h):
    bm = 400
    grid = (N // bm,)
    return pl.pallas_call(
        _comb_body,
        grid=grid,
        in_specs=[
            pl.BlockSpec((NC, bm, D), lambda i: (0, i, 0)),
            pl.BlockSpec((NC, bm, 16), lambda i: (0, i, 0)),
            pl.BlockSpec((16, D), lambda i: (0, 0)),
            pl.BlockSpec((1, D), lambda i: (0, 0)),
            pl.BlockSpec((bm, D), lambda i: (i, 0)),
        ],
        out_specs=pl.BlockSpec((bm, D), lambda i: (i, 0)),
        out_shape=jax.ShapeDtypeStruct((N, D), _f32),
    )(pnum, pden, rep, b2d, h)


def kernel(x, edge_index, edge_indice, edge_type, edge_dialog,
           W1, A1, b1, W2, A2, b2):
    src = edge_index[0]
    dst = edge_index[1]
    npad = EPAD - E
    srcp = jnp.concatenate([src, jnp.zeros((npad,), _i32)])
    dstp = jnp.concatenate([dst, jnp.full((npad,), N, _i32)])

    mask = jnp.kron(jnp.eye(H, dtype=_f32), jnp.ones((DH, 1), _f32))
    mask = jnp.concatenate([mask, jnp.zeros((D, 16 - H), _f32)], axis=1)
    rep = jnp.concatenate(
        [jnp.kron(jnp.eye(H, dtype=_f32), jnp.ones((1, DH), _f32)),
         jnp.zeros((16 - H, D), _f32)], axis=0)

    h = x
    for (W, A, b) in ((W1, A1, b1), (W2, A2, b2)):
        wcat = W.transpose(1, 0, 2).reshape(D, D)
        ssrc = mask * A[:, :DH, 0].reshape(D, 1)
        sdst = mask * A[:, DH:, 0].reshape(D, 1)
        xf, as_t, ad_t = _prep(h, wcat, ssrc, sdst)
        g16 = jnp.max(as_t, axis=0)
        ad_tp = jnp.concatenate([ad_t, jnp.zeros((1, 16), _f32)])
        pnum, pden = _edge_sc_build()(
            xf, as_t, ad_tp, srcp, dstp, g16,
            jnp.zeros((K, D), _f32), jnp.zeros((K, 16), _f32))
        h = _combine(pnum, pden, rep, b.reshape(1, D), h)
    return h


# speedup vs baseline: 37.8880x; 1.3707x over previous
import functools

import jax
import jax.numpy as jnp
from jax import lax
from jax.experimental import pallas as pl
from jax.experimental.pallas import tpu as pltpu
from jax.experimental.pallas import tpu_sc as plsc

N = 10000
E = 320000
D = 128
H = 4
DH = 32
DW = D + 16

NC = 2
NS = 16
NW = NC * NS
K = 128
NB = 80
EW = NB * K
EPAD = NW * EW
NACC = N + 1

_f32 = jnp.float32
_i32 = jnp.int32


def _prep_body(h_ref, wc_ref, ss_ref, sd_ref, xfa_ref, ad_ref):
    xfb = jax.lax.dot_general(
        h_ref[...], wc_ref[...], (((1,), (0,)), ((), ())),
        precision=jax.lax.Precision.HIGHEST, preferred_element_type=_f32)
    xfa_ref[:, pl.ds(0, D)] = xfb
    xfa_ref[:, pl.ds(D, 16)] = jax.lax.dot_general(
        xfb, ss_ref[...], (((1,), (0,)), ((), ())),
        precision=jax.lax.Precision.HIGHEST, preferred_element_type=_f32)
    ad_ref[...] = jax.lax.dot_general(
        xfb, sd_ref[...], (((1,), (0,)), ((), ())),
        precision=jax.lax.Precision.HIGHEST, preferred_element_type=_f32)


def _prep(h, wcat, ssrc, sdst):
    bm = 400
    grid = (N // bm,)
    return pl.pallas_call(
        _prep_body,
        grid=grid,
        in_specs=[
            pl.BlockSpec((bm, D), lambda i: (i, 0)),
            pl.BlockSpec((D, D), lambda i: (0, 0)),
            pl.BlockSpec((D, 16), lambda i: (0, 0)),
            pl.BlockSpec((D, 16), lambda i: (0, 0)),
        ],
        out_specs=[
            pl.BlockSpec((bm, DW), lambda i: (i, 0)),
            pl.BlockSpec((bm, 16), lambda i: (i, 0)),
        ],
        out_shape=[
            jax.ShapeDtypeStruct((N, DW), _f32),
            jax.ShapeDtypeStruct((N, 16), _f32),
        ],
    )(h, wcat, ssrc, sdst)


def _edge_body(xfa_hbm, ad_hbm, src_hbm, dst_hbm, g_hbm, z_hbm, p_hbm,
               acc, src_i0, src_i1, dst_i0, dst_i1, b0, b1, ad_b,
               g_v, sem0, sem1):
    c = lax.axis_index("c")
    s = lax.axis_index("s")
    wid = c * NS + s

    iota = lax.iota(_i32, 16)
    headmask = iota < H
    zeros16 = jnp.zeros((16,), _f32)

    src_i = (src_i0, src_i1)
    dst_i = (dst_i0, dst_i1)
    bufs = (b0, b1)
    sems = (sem0, sem1)

    nz = (NACC + K - 1) // K
    for k in range(5):
        t = s + NS * k

        @pl.when(t < nz)
        def _():
            start = jnp.minimum(t * K, NACC - K)
            pltpu.sync_copy(z_hbm, acc.at[pl.ds(start, K)])

    pltpu.sync_copy(g_hbm, g_v)
    plsc.subcore_barrier()

    gv = g_v[...]

    def _prefetch(p, blk):
        base = wid * EW + blk * K
        pltpu.sync_copy(src_hbm.at[pl.ds(base, K)], src_i[p])
        pltpu.sync_copy(dst_hbm.at[pl.ds(base, K)], dst_i[p])
        pltpu.async_copy(xfa_hbm.at[src_i[p]], bufs[p], sems[p])

    def _consume(p):
        pltpu.sync_copy(ad_hbm.at[dst_i[p]], ad_b)
        pltpu.make_async_copy(xfa_hbm.at[src_i[p]], bufs[p], sems[p]).wait()
        buf = bufs[p]

        def _edge(e, _):
            asv = buf[e, pl.ds(D, 16)]
            adv = ad_b[e, :]
            sv = asv + adv
            lr = jnp.maximum(sv, 0.2 * sv)
            tv = gv + adv
            mv = jnp.maximum(tv, 0.2 * tv)
            wv = jnp.where(headmask, jnp.exp(lr - mv), zeros16)
            buf[e, pl.ds(D, 16)] = wv
            for h in range(H):
                wb = wv.at[jnp.full((16,), h, _i32)].get(
                    mode="promise_in_bounds")
                for half in range(2):
                    j = (h * 2 + half) * 16
                    buf[e, pl.ds(j, 16)] = buf[e, pl.ds(j, 16)] * wb
            return 0
        lax.fori_loop(0, K, _edge, 0)
        pltpu.sync_copy(buf, acc.at[dst_i[p]], add=True)

    _prefetch(0, 0)

    def _pair(i, _):
        _prefetch(1, 2 * i + 1)
        _consume(0)
        _prefetch(0, 2 * i + 2)
        _consume(1)
        return 0
    lax.fori_loop(0, NB // 2, _pair, 0)

    pltpu.make_async_copy(xfa_hbm.at[src_i[0]], bufs[0], sems[0]).wait()

    plsc.subcore_barrier()

    nd = (N + K - 1) // K
    for k in range(5):
        t = s + NS * k

        @pl.when(t < nd)
        def _():
            start = jnp.minimum(t * K, N - K)
            pltpu.sync_copy(acc.at[pl.ds(start, K)],
                            p_hbm.at[c, pl.ds(start, K)])


@functools.cache
def _edge_sc_build():
    return pl.kernel(
        _edge_body,
        out_type=[
            jax.ShapeDtypeStruct((NC, N, DW), _f32),
        ],
        mesh=plsc.VectorSubcoreMesh(core_axis_name="c", subcore_axis_name="s",
                                    num_cores=NC, num_subcores=NS),
        compiler_params=pltpu.CompilerParams(use_tc_tiling_on_sc=False),
        scratch_types=[
            pltpu.VMEM_SHARED((NACC, DW), _f32),
            pltpu.VMEM((K,), _i32),
            pltpu.VMEM((K,), _i32),
            pltpu.VMEM((K,), _i32),
            pltpu.VMEM((K,), _i32),
            pltpu.VMEM((K, DW), _f32),
            pltpu.VMEM((K, DW), _f32),
            pltpu.VMEM((K, 16), _f32),
            pltpu.VMEM((16,), _f32),
            pltpu.SemaphoreType.DMA,
            pltpu.SemaphoreType.DMA,
        ],
    )


def _comb_body(p_ref, rep_ref, b_ref, h_ref, out_ref):
    num = p_ref[0, :, pl.ds(0, D)] + p_ref[1, :, pl.ds(0, D)]
    den = p_ref[0, :, pl.ds(D, 16)] + p_ref[1, :, pl.ds(D, 16)]
    denx = jax.lax.dot_general(
        den, rep_ref[...], (((1,), (0,)), ((), ())),
        precision=jax.lax.Precision.HIGHEST, preferred_element_type=_f32)
    att = num / (denx + 1e-16) + b_ref[...]
    out_ref[...] = h_ref[...] + jnp.where(
        att > 0, att, jnp.exp(jnp.minimum(att, 0.0)) - 1.0)


def _combine(p, rep, b2d, h):
    bm = 400
    grid = (N // bm,)
    return pl.pallas_call(
        _comb_body,
        grid=grid,
        in_specs=[
            pl.BlockSpec((NC, bm, DW), lambda i: (0, i, 0)),
            pl.BlockSpec((16, D), lambda i: (0, 0)),
            pl.BlockSpec((1, D), lambda i: (0, 0)),
            pl.BlockSpec((bm, D), lambda i: (i, 0)),
        ],
        out_specs=pl.BlockSpec((bm, D), lambda i: (i, 0)),
        out_shape=jax.ShapeDtypeStruct((N, D), _f32),
    )(p, rep, b2d, h)


def kernel(x, edge_index, edge_indice, edge_type, edge_dialog,
           W1, A1, b1, W2, A2, b2):
    src = edge_index[0]
    dst = edge_index[1]
    npad = EPAD + K - E
    srcp = jnp.concatenate([src, jnp.zeros((npad,), _i32)])
    dstp = jnp.concatenate([dst, jnp.full((npad,), N, _i32)])

    mask = jnp.kron(jnp.eye(H, dtype=_f32), jnp.ones((DH, 1), _f32))
    mask = jnp.concatenate([mask, jnp.zeros((D, 16 - H), _f32)], axis=1)
    rep = jnp.concatenate(
        [jnp.kron(jnp.eye(H, dtype=_f32), jnp.ones((1, DH), _f32)),
         jnp.zeros((16 - H, D), _f32)], axis=0)

    h = x
    for (W, A, b) in ((W1, A1, b1), (W2, A2, b2)):
        wcat = W.transpose(1, 0, 2).reshape(D, D)
        ssrc = mask * A[:, :DH, 0].reshape(D, 1)
        sdst = mask * A[:, DH:, 0].reshape(D, 1)
        xfa, ad_t = _prep(h, wcat, ssrc, sdst)
        g16 = jnp.max(xfa[:, D:], axis=0)
        ad_tp = jnp.concatenate([ad_t, jnp.zeros((1, 16), _f32)])
        (part,) = _edge_sc_build()(
            xfa, ad_tp, srcp, dstp, g16, jnp.zeros((K, DW), _f32))
        h = _combine(part, rep, b.reshape(1, D), h)
    return h


# speedup vs baseline: 41.9206x; 1.1064x over previous
import functools

import jax
import jax.numpy as jnp
from jax import lax
from jax.experimental import pallas as pl
from jax.experimental.pallas import tpu as pltpu
from jax.experimental.pallas import tpu_sc as plsc

N = 10000
E = 320000
D = 128
H = 4
DH = 32
DW = D + 16

NC = 2
NS = 16
NW = NC * NS
K = 128
NB = 80
EW = NB * K
EPAD = NW * EW
NACC = N + 1

_f32 = jnp.float32
_i32 = jnp.int32


def _prep_body(h_ref, wc_ref, ss_ref, sd_ref, xfa_ref, ad_ref):
    xfb = jax.lax.dot_general(
        h_ref[...], wc_ref[...], (((1,), (0,)), ((), ())),
        precision=jax.lax.Precision.HIGHEST, preferred_element_type=_f32)
    xfa_ref[:, pl.ds(0, D)] = xfb
    xfa_ref[:, pl.ds(D, 16)] = jax.lax.dot_general(
        xfb, ss_ref[...], (((1,), (0,)), ((), ())),
        precision=jax.lax.Precision.HIGHEST, preferred_element_type=_f32)
    ad_ref[...] = jax.lax.dot_general(
        xfb, sd_ref[...], (((1,), (0,)), ((), ())),
        precision=jax.lax.Precision.HIGHEST, preferred_element_type=_f32)


def _prep(h, wcat, ssrc, sdst):
    bm = 400
    grid = (N // bm,)
    return pl.pallas_call(
        _prep_body,
        grid=grid,
        in_specs=[
            pl.BlockSpec((bm, D), lambda i: (i, 0)),
            pl.BlockSpec((D, D), lambda i: (0, 0)),
            pl.BlockSpec((D, 16), lambda i: (0, 0)),
            pl.BlockSpec((D, 16), lambda i: (0, 0)),
        ],
        out_specs=[
            pl.BlockSpec((bm, DW), lambda i: (i, 0)),
            pl.BlockSpec((bm, 16), lambda i: (i, 0)),
        ],
        out_shape=[
            jax.ShapeDtypeStruct((N, DW), _f32),
            jax.ShapeDtypeStruct((N, 16), _f32),
        ],
    )(h, wcat, ssrc, sdst)


def _edge_body(xfa_hbm, ad_hbm, src_hbm, dst_hbm, g_hbm, z_hbm, p_hbm,
               acc, src_i0, src_i1, dst_i0, dst_i1, b0, b1, ad_b,
               g_v, sem0, sem1):
    c = lax.axis_index("c")
    s = lax.axis_index("s")
    wid = c * NS + s

    iota = lax.iota(_i32, 16)
    headmask = iota < H
    zeros16 = jnp.zeros((16,), _f32)

    src_i = (src_i0, src_i1)
    dst_i = (dst_i0, dst_i1)
    bufs = (b0, b1)
    sems = (sem0, sem1)

    nz = (NACC + K - 1) // K
    for k in range(5):
        t = s + NS * k

        @pl.when(t < nz)
        def _():
            start = jnp.minimum(t * K, NACC - K)
            pltpu.sync_copy(z_hbm, acc.at[pl.ds(start, K)])

    pltpu.sync_copy(g_hbm, g_v)
    plsc.subcore_barrier()

    gv = g_v[...]

    def _prefetch(p, blk):
        base = wid * EW + blk * K
        pltpu.sync_copy(src_hbm.at[pl.ds(base, K)], src_i[p])
        pltpu.sync_copy(dst_hbm.at[pl.ds(base, K)], dst_i[p])
        pltpu.async_copy(xfa_hbm.at[src_i[p]], bufs[p], sems[p])

    def _consume(p):
        pltpu.sync_copy(ad_hbm.at[dst_i[p]], ad_b)
        pltpu.make_async_copy(xfa_hbm.at[src_i[p]], bufs[p], sems[p]).wait()
        buf = bufs[p]

        @plsc.parallel_loop(0, K, 1, unroll=4)
        def _edge(e):
            asv = buf[e, pl.ds(D, 16)]
            adv = ad_b[e, :]
            sv = asv + adv
            lr = jnp.maximum(sv, 0.2 * sv)
            tv = gv + adv
            mv = jnp.maximum(tv, 0.2 * tv)
            wv = jnp.where(headmask, jnp.exp(lr - mv), zeros16)
            buf[e, pl.ds(D, 16)] = wv
            for h in range(H):
                wb = wv.at[jnp.full((16,), h, _i32)].get(
                    mode="promise_in_bounds")
                for half in range(2):
                    j = (h * 2 + half) * 16
                    buf[e, pl.ds(j, 16)] = buf[e, pl.ds(j, 16)] * wb
        pltpu.sync_copy(buf, acc.at[dst_i[p]], add=True)

    _prefetch(0, 0)

    def _pair(i, _):
        _prefetch(1, 2 * i + 1)
        _consume(0)
        _prefetch(0, 2 * i + 2)
        _consume(1)
        return 0
    lax.fori_loop(0, NB // 2, _pair, 0)

    pltpu.make_async_copy(xfa_hbm.at[src_i[0]], bufs[0], sems[0]).wait()

    plsc.subcore_barrier()

    nd = (N + K - 1) // K
    for k in range(5):
        t = s + NS * k

        @pl.when(t < nd)
        def _():
            start = jnp.minimum(t * K, N - K)
            pltpu.sync_copy(acc.at[pl.ds(start, K)],
                            p_hbm.at[c, pl.ds(start, K)])


@functools.cache
def _edge_sc_build():
    return pl.kernel(
        _edge_body,
        out_type=[
            jax.ShapeDtypeStruct((NC, N, DW), _f32),
        ],
        mesh=plsc.VectorSubcoreMesh(core_axis_name="c", subcore_axis_name="s",
                                    num_cores=NC, num_subcores=NS),
        compiler_params=pltpu.CompilerParams(use_tc_tiling_on_sc=False),
        scratch_types=[
            pltpu.VMEM_SHARED((NACC, DW), _f32),
            pltpu.VMEM((K,), _i32),
            pltpu.VMEM((K,), _i32),
            pltpu.VMEM((K,), _i32),
            pltpu.VMEM((K,), _i32),
            pltpu.VMEM((K, DW), _f32),
            pltpu.VMEM((K, DW), _f32),
            pltpu.VMEM((K, 16), _f32),
            pltpu.VMEM((16,), _f32),
            pltpu.SemaphoreType.DMA,
            pltpu.SemaphoreType.DMA,
        ],
    )


def _comb_body(p_ref, rep_ref, b_ref, h_ref, out_ref):
    num = p_ref[0, :, pl.ds(0, D)] + p_ref[1, :, pl.ds(0, D)]
    den = p_ref[0, :, pl.ds(D, 16)] + p_ref[1, :, pl.ds(D, 16)]
    denx = jax.lax.dot_general(
        den, rep_ref[...], (((1,), (0,)), ((), ())),
        precision=jax.lax.Precision.HIGHEST, preferred_element_type=_f32)
    att = num / (denx + 1e-16) + b_ref[...]
    out_ref[...] = h_ref[...] + jnp.where(
        att > 0, att, jnp.exp(jnp.minimum(att, 0.0)) - 1.0)


def _combine(p, rep, b2d, h):
    bm = 400
    grid = (N // bm,)
    return pl.pallas_call(
        _comb_body,
        grid=grid,
        in_specs=[
            pl.BlockSpec((NC, bm, DW), lambda i: (0, i, 0)),
            pl.BlockSpec((16, D), lambda i: (0, 0)),
            pl.BlockSpec((1, D), lambda i: (0, 0)),
            pl.BlockSpec((bm, D), lambda i: (i, 0)),
        ],
        out_specs=pl.BlockSpec((bm, D), lambda i: (i, 0)),
        out_shape=jax.ShapeDtypeStruct((N, D), _f32),
    )(p, rep, b2d, h)


def kernel(x, edge_index, edge_indice, edge_type, edge_dialog,
           W1, A1, b1, W2, A2, b2):
    src = edge_index[0]
    dst = edge_index[1]
    npad = EPAD + K - E
    srcp = jnp.concatenate([src, jnp.zeros((npad,), _i32)])
    dstp = jnp.concatenate([dst, jnp.full((npad,), N, _i32)])

    mask = jnp.kron(jnp.eye(H, dtype=_f32), jnp.ones((DH, 1), _f32))
    mask = jnp.concatenate([mask, jnp.zeros((D, 16 - H), _f32)], axis=1)
    rep = jnp.concatenate(
        [jnp.kron(jnp.eye(H, dtype=_f32), jnp.ones((1, DH), _f32)),
         jnp.zeros((16 - H, D), _f32)], axis=0)

    h = x
    for (W, A, b) in ((W1, A1, b1), (W2, A2, b2)):
        wcat = W.transpose(1, 0, 2).reshape(D, D)
        ssrc = mask * A[:, :DH, 0].reshape(D, 1)
        sdst = mask * A[:, DH:, 0].reshape(D, 1)
        xfa, ad_t = _prep(h, wcat, ssrc, sdst)
        g16 = jnp.max(xfa[:, D:], axis=0)
        ad_tp = jnp.concatenate([ad_t, jnp.zeros((1, 16), _f32)])
        (part,) = _edge_sc_build()(
            xfa, ad_tp, srcp, dstp, g16, jnp.zeros((K, DW), _f32))
        h = _combine(part, rep, b.reshape(1, D), h)
    return h
